# baseline (speedup 1.0000x reference)
import functools
import jax
import jax.numpy as jnp
from jax import lax
from jax.experimental import pallas as pl
from jax.experimental.pallas import tpu as pltpu
from jax.experimental.pallas import tpu_sc as plsc

N_NODES = 10000
F_IN = 128
HID = 128
NCLASS = 16

NC = 2
NS = 16
NW = NC * NS
CHUNK = 128
N_PAD = 10240
ROWS_PER_TILE = N_PAD // NS


def _mm_body(x_ref, w_ref, o_ref):
    o_ref[...] = jnp.dot(x_ref[...], w_ref[...],
                         preferred_element_type=jnp.float32)


def _dense_matmul(x, w):
    return pl.pallas_call(
        _mm_body,
        out_shape=jax.ShapeDtypeStruct((x.shape[0], w.shape[1]), jnp.float32),
    )(x, w)


def _spmm_body(h_hbm, src_hbm, dst_hbm, w_hbm, acc_out, c_out,
               src_v, dst_v, w_v, rows, acc_sh, c_sh, csem):
    cid = lax.axis_index("c")
    sid = lax.axis_index("s")
    n_chunks = src_v.shape[0]

    def zero_row(r, _):
        for f in range(8):
            rows[r, pl.ds(f * 16, 16)] = jnp.zeros((16,), jnp.float32)
        return _
    lax.fori_loop(0, CHUNK, zero_row, None)
    for t in range(ROWS_PER_TILE // CHUNK):
        off = sid * ROWS_PER_TILE + t * CHUNK
        pltpu.sync_copy(rows, acc_sh.at[pl.ds(off, CHUNK)])
        pltpu.sync_copy(rows.at[0], c_sh.at[pl.ds(off, CHUNK)])
    plsc.subcore_barrier()

    pltpu.sync_copy(src_hbm.at[cid, sid], src_v)
    pltpu.sync_copy(dst_hbm.at[cid, sid], dst_v)
    pltpu.sync_copy(w_hbm.at[cid, sid], w_v)

    def cscat(j):
        return pltpu.make_async_copy(w_v.at[j], c_sh.at[src_v.at[j]], csem)

    def process(j):
        pltpu.sync_copy(h_hbm.at[src_v.at[j]], rows)

        def scale_block(b, __):
            wvec = w_v[j, pl.ds(b * 16, 16)]
            for l in range(16):
                i = b * 16 + l
                wb = jnp.full((16,), wvec[l], jnp.float32)
                for f in range(8):
                    sl = pl.ds(f * 16, 16)
                    rows[i, sl] = rows[i, sl] * wb
            return __
        lax.fori_loop(0, CHUNK // 16, scale_block, None)

        pltpu.sync_copy(rows, acc_sh.at[dst_v.at[j]], add=True)

    process(0)
    cscat(0).start(add=True)

    def edge_chunk(j, _):
        process(j)
        cscat(0).wait()
        cscat(j).start(add=True)
        return _

    lax.fori_loop(1, n_chunks, edge_chunk, None)
    cscat(0).wait()
    plsc.subcore_barrier()

    off = sid * ROWS_PER_TILE
    pltpu.sync_copy(acc_sh.at[pl.ds(off, ROWS_PER_TILE)],
                    acc_out.at[cid, pl.ds(off, ROWS_PER_TILE)])
    pltpu.sync_copy(c_sh.at[pl.ds(off, ROWS_PER_TILE)],
                    c_out.at[cid, pl.ds(off, ROWS_PER_TILE)])


def _spmm(h, src4, dst4, w4):
    n_max = src4.shape[2]
    f = h.shape[1]
    kern = functools.partial(
        pl.kernel,
        out_type=(
            jax.ShapeDtypeStruct((NC, N_PAD, f), jnp.float32),
            jax.ShapeDtypeStruct((NC, N_PAD), jnp.float32),
        ),
        mesh=plsc.VectorSubcoreMesh(core_axis_name="c", subcore_axis_name="s"),
        scratch_types=[
            pltpu.VMEM((n_max, CHUNK), jnp.int32),
            pltpu.VMEM((n_max, CHUNK), jnp.int32),
            pltpu.VMEM((n_max, CHUNK), jnp.float32),
            pltpu.VMEM((CHUNK, f), jnp.float32),
            pltpu.VMEM_SHARED((N_PAD, f), jnp.float32),
            pltpu.VMEM_SHARED((N_PAD,), jnp.float32),
            pltpu.SemaphoreType.DMA,
        ],
    )(_spmm_body)
    return kern(h, src4, dst4, w4)


def _reduce_body(a0_ref, a1_ref, c0_ref, c1_ref, w2_ref, o_ref):
    i = pl.program_id(0)
    h1 = jnp.maximum(a0_ref[...] + a1_ref[...], 0.0)
    s = jnp.sum(h1 * (c0_ref[...] + c1_ref[...]), axis=0)[None, :]
    val = jnp.dot(s, w2_ref[...],
                  preferred_element_type=jnp.float32) * (1.0 / N_NODES)

    @pl.when(i == 0)
    def _():
        o_ref[...] = val

    @pl.when(i > 0)
    def _():
        o_ref[...] = o_ref[...] + val


def _reduce(acc, c, w2):
    blk = 1024
    grid = N_PAD // blk
    return pl.pallas_call(
        _reduce_body,
        grid=(grid,),
        in_specs=[
            pl.BlockSpec((blk, HID), lambda i: (i, 0)),
            pl.BlockSpec((blk, HID), lambda i: (i, 0)),
            pl.BlockSpec((blk, 1), lambda i: (i, 0)),
            pl.BlockSpec((blk, 1), lambda i: (i, 0)),
            pl.BlockSpec((HID, NCLASS), lambda i: (0, 0)),
        ],
        out_specs=pl.BlockSpec((1, NCLASS), lambda i: (0, 0)),
        out_shape=jax.ShapeDtypeStruct((1, NCLASS), jnp.float32),
    )(acc[0], acc[1], c[0].reshape(N_PAD, 1), c[1].reshape(N_PAD, 1), w2)


def kernel(x, edge_index, edge_weight, W1, W2):
    e = edge_weight.shape[0]
    per_tile = -(-e // (NW * CHUNK)) * CHUNK
    e_pad = per_tile * NW
    n_chunks = per_tile // CHUNK

    src = jnp.asarray(edge_index[0], jnp.int32)
    dst = jnp.asarray(edge_index[1], jnp.int32)
    w = jnp.asarray(edge_weight, jnp.float32)

    def split(arr):
        return jnp.pad(arr, (0, e_pad - e)).reshape(
            NS, NC, n_chunks, CHUNK).transpose(1, 0, 2, 3)

    h = _dense_matmul(x, W1)
    acc, c = _spmm(h, split(src), split(dst), split(w))
    return _reduce(acc, c, W2)
